# baseline (speedup 1.0000x reference)
import functools

import jax
import jax.numpy as jnp
from jax import lax
from jax.experimental import pallas as pl
from jax.experimental.pallas import tpu as pltpu
from jax.experimental.pallas import tpu_sc as plsc

N = 10000
E = 320000
D = 128
CW = 16
DP = D + CW

NC = 2
NS = 16
NW = NC * NS
EPW = E // NW
C = 80
NCHUNK = EPW // C
SG = 5
NSG = NCHUNK // SG
RPT = 624
RPT_LAST = N - (NS - 1) * RPT
CK = 16


def _make_sc_agg(dp: int, ring: int):
  mesh = plsc.VectorSubcoreMesh(core_axis_name="c", subcore_axis_name="s",
                                num_cores=NC, num_subcores=NS)
  out_type = [jax.ShapeDtypeStruct((NC, N, dp), jnp.float32)]
  scratch = (
      [pltpu.VMEM_SHARED((N, dp), jnp.float32)]
      + [pltpu.VMEM((SG, C), jnp.int32)] * 4
      + [pltpu.VMEM((C, dp), jnp.float32)] * ring
      + [pltpu.SemaphoreType.DMA] * (2 + 2 * ring)
  )
  unit = 10 * ring // (2 if ring % 2 == 0 else 1)
  nit = (NCHUNK - SG) // unit

  def body(feat_hbm, ei_hbm, agg_out, *rest):
    agg_sh = rest[0]
    srci = rest[1:3]
    dsti = rest[3:5]
    rows = rest[5:5 + ring]
    si = rest[5 + ring:7 + ring]
    sgs = rest[7 + ring:7 + 2 * ring]
    sss = rest[7 + 2 * ring:7 + 3 * ring]
    cid = lax.axis_index("c")
    sid = lax.axis_index("s")
    wid = cid * NS + sid

    roff = pl.multiple_of(sid * RPT, 8)
    nch = jnp.where(sid < NS - 1, RPT // CK, RPT_LAST // CK)

    zvec = jnp.zeros((16,), jnp.float32)
    for r in range(CK):
      for j in range(dp // 16):
        rows[0][r, pl.ds(j * 16, 16)] = zvec

    def zstep(i, carry):
      ro = pl.multiple_of(roff + i * CK, 8)
      pltpu.sync_copy(rows[0].at[pl.ds(0, CK)], agg_sh.at[pl.ds(ro, CK)])
      return carry

    lax.fori_loop(0, nch, zstep, 0)

    gdst = rows

    def _slab(p, g, start):
      base = wid * EPW + g * (SG * C)
      for k in range(SG):
        for row, buf in ((0, srci[p]), (1, dsti[p])):
          d = pltpu.make_async_copy(
              ei_hbm.at[row, pl.ds(base + k * C, C)], buf.at[k], si[p])
          d.start() if start else d.wait()

    def slab_start(p, g):
      _slab(p, g, True)

    def slab_wait(p, g):
      _slab(p, g, False)

    def gather_start(b, p, k):
      pltpu.async_copy(feat_hbm.at[srci[p].at[k]], gdst[b], sgs[b])

    def gather_wait(b, p, k):
      pltpu.make_async_copy(feat_hbm.at[srci[p].at[k]], gdst[b], sgs[b]).wait()

    def scatter_start(b, p, k):
      pltpu.async_copy(rows[b], agg_sh.at[dsti[p].at[k]], sss[b], add=True)

    def scatter_drain(b):
      pltpu.make_async_copy(rows[b], agg_sh.at[dsti[0].at[0]], sss[b]).wait()

    def chunk_ops(c, g_of, drain, first_group):
      k, lg, b = c % SG, c // SG, (c if first_group else SG + c) % ring
      p = (lg if first_group else 1 + lg) % 2
      if drain:
        scatter_drain(b)
      if k == 0:
        slab_wait(p, g_of(lg))
      gather_start(b, p, k)
      c2 = c - 1
      if c2 >= 0 or not first_group:
        k2, lg2 = c2 % SG, c2 // SG
        b2 = (c2 if first_group else SG + c2) % ring
        p2 = (lg2 if first_group else 1 + lg2) % 2
        gather_wait(b2, p2, k2)
        scatter_start(b2, p2, k2)
      if k == ring - 1:
        slab_start(1 - p, g_of(lg + 1))

    slab_start(0, 0)
    plsc.subcore_barrier()

    for c in range(SG):
      chunk_ops(c, lambda lg: jnp.int32(lg), drain=(c >= ring),
                first_group=True)

    gpi = unit // SG

    def step(i, carry):
      def g_of(lg):
        return jnp.minimum(1 + gpi * i + lg, NSG - 1)
      for c in range(unit):
        chunk_ops(c, g_of, drain=True, first_group=False)
      return carry

    lax.fori_loop(0, nit, step, 0)

    for t in (NCHUNK - 1,):
      k2, b2 = t % SG, t % ring
      gather_wait(b2, 0, k2)
      scatter_start(b2, 0, k2)
    for t in range(NCHUNK - ring, NCHUNK):
      scatter_drain(t % ring)
    slab_wait(1, NSG - 1)
    plsc.subcore_barrier()

    def wstep(i, carry):
      ro = pl.multiple_of(roff + i * CK, 8)
      pltpu.sync_copy(agg_sh.at[pl.ds(ro, CK)], rows[0].at[pl.ds(0, CK)])
      pltpu.sync_copy(rows[0].at[pl.ds(0, CK)], agg_out.at[cid, pl.ds(ro, CK)])
      return carry

    lax.fori_loop(0, nch, wstep, 0)

  return pl.kernel(
      body, out_type=out_type, mesh=mesh, scratch_types=scratch,
      compiler_params=pltpu.CompilerParams(use_tc_tiling_on_sc=False))


_sc_agg_cnt = _make_sc_agg(DP, 3)
_sc_agg = _make_sc_agg(D, 4)

BN = 1000


def _dense1_body(agg_ref, x_ref, wl_ref, b_ref, wr_ref, out_ref):
  a = agg_ref[0] + agg_ref[1]
  cnt = a[:, D:D + 1]
  mean = a[:, :D] / jnp.maximum(cnt, 1.0)
  h = (lax.dot_general(mean, wl_ref[...], (((1,), (1,)), ((), ())),
                       preferred_element_type=jnp.float32)
       + b_ref[...]
       + lax.dot_general(x_ref[...], wr_ref[...], (((1,), (1,)), ((), ())),
                         preferred_element_type=jnp.float32))
  nrm = jnp.sqrt(jnp.sum(h * h, axis=-1, keepdims=True))
  h = h / jnp.maximum(nrm, 1e-12)
  out_ref[...] = jnp.maximum(h, 0.0)


def _dense2_body(agg_ref, cnt_ref, x_ref, wl_ref, b_ref, wr_ref, out_ref):
  a = agg_ref[0] + agg_ref[1]
  cnt = cnt_ref[0, :, 0:1] + cnt_ref[1, :, 0:1]
  mean = a / jnp.maximum(cnt, 1.0)
  h = (lax.dot_general(mean, wl_ref[...], (((1,), (1,)), ((), ())),
                       preferred_element_type=jnp.float32)
       + b_ref[...]
       + lax.dot_general(x_ref[...], wr_ref[...], (((1,), (1,)), ((), ())),
                         preferred_element_type=jnp.float32))
  nrm = jnp.sqrt(jnp.sum(h * h, axis=-1, keepdims=True))
  out_ref[...] = h / jnp.maximum(nrm, 1e-12)


def kernel(x, edge_index, W1l, b1, W1r, W2l, b2, W2r):
  xp = jnp.concatenate([x, jnp.ones((N, CW), jnp.float32)], axis=1)
  (aggx,) = _sc_agg_cnt(xp, edge_index)
  wspec = pl.BlockSpec((D, D), lambda i: (0, 0))
  bspec = pl.BlockSpec((1, D), lambda i: (0, 0))
  h1 = pl.pallas_call(
      _dense1_body,
      grid=(N // BN,),
      in_specs=[
          pl.BlockSpec((NC, BN, DP), lambda i: (0, i, 0)),
          pl.BlockSpec((BN, D), lambda i: (i, 0)),
          wspec, bspec, wspec,
      ],
      out_specs=pl.BlockSpec((BN, D), lambda i: (i, 0)),
      out_shape=jax.ShapeDtypeStruct((N, D), jnp.float32),
  )(aggx, x, W1l, b1.reshape(1, D), W1r)
  (agg2,) = _sc_agg(h1, edge_index)
  return pl.pallas_call(
      _dense2_body,
      grid=(N // BN,),
      in_specs=[
          pl.BlockSpec((NC, BN, D), lambda i: (0, i, 0)),
          pl.BlockSpec((NC, BN, CW), lambda i: (0, i, 0)),
          pl.BlockSpec((BN, D), lambda i: (i, 0)),
          wspec, bspec, wspec,
      ],
      out_specs=pl.BlockSpec((BN, D), lambda i: (i, 0)),
      out_shape=jax.ShapeDtypeStruct((N, D), jnp.float32),
  )(agg2, aggx[:, :, D:], h1, W2l, b2.reshape(1, D), W2r)


# speedup vs baseline: 1.0381x; 1.0381x over previous
import functools

import jax
import jax.numpy as jnp
from jax import lax
from jax.experimental import pallas as pl
from jax.experimental.pallas import tpu as pltpu
from jax.experimental.pallas import tpu_sc as plsc

N = 10000
E = 320000
D = 128
CW = 16
DP = D + CW

NC = 2
NS = 16
NW = NC * NS
EPW = E // NW
C = 80
NCHUNK = EPW // C
SG = 5
NSG = NCHUNK // SG
RPT = 624
RPT_LAST = N - (NS - 1) * RPT
CK = 16


def _make_sc_agg(dp: int, ring: int):
  mesh = plsc.VectorSubcoreMesh(core_axis_name="c", subcore_axis_name="s",
                                num_cores=NC, num_subcores=NS)
  out_type = [jax.ShapeDtypeStruct((NC, N, dp), jnp.float32)]
  scratch = (
      [pltpu.VMEM_SHARED((N, dp), jnp.float32)]
      + [pltpu.VMEM((SG, C), jnp.int32)] * 4
      + [pltpu.VMEM((C, dp), jnp.float32)] * ring
      + [pltpu.SemaphoreType.DMA] * (2 + 2 * ring)
  )
  unit = 10 * ring // (2 if ring % 2 == 0 else 1)
  nit = (NCHUNK - SG) // unit

  def body(feat_hbm, ei_hbm, agg_out, *rest):
    agg_sh = rest[0]
    srci = rest[1:3]
    dsti = rest[3:5]
    rows = rest[5:5 + ring]
    si = rest[5 + ring:7 + ring]
    sgs = rest[7 + ring:7 + 2 * ring]
    sss = rest[7 + 2 * ring:7 + 3 * ring]
    cid = lax.axis_index("c")
    sid = lax.axis_index("s")
    wid = cid * NS + sid

    roff = pl.multiple_of(sid * RPT, 8)
    nch = jnp.where(sid < NS - 1, RPT // CK, RPT_LAST // CK)

    zvec = jnp.zeros((16,), jnp.float32)
    for r in range(CK):
      for j in range(dp // 16):
        rows[0][r, pl.ds(j * 16, 16)] = zvec

    def zstep(i, carry):
      ro = pl.multiple_of(roff + i * CK, 8)
      pltpu.sync_copy(rows[0].at[pl.ds(0, CK)], agg_sh.at[pl.ds(ro, CK)])
      return carry

    lax.fori_loop(0, nch, zstep, 0)

    gdst = rows

    def _slab(p, g, start):
      base = wid * EPW + g * (SG * C)
      for k in range(SG):
        for row, buf in ((0, srci[p]), (1, dsti[p])):
          d = pltpu.make_async_copy(
              ei_hbm.at[row, pl.ds(base + k * C, C)], buf.at[k], si[p])
          d.start() if start else d.wait()

    def slab_start(p, g):
      _slab(p, g, True)

    def slab_wait(p, g):
      _slab(p, g, False)

    def gather_start(b, p, k):
      pltpu.async_copy(feat_hbm.at[srci[p].at[k]], gdst[b], sgs[b])

    def gather_wait(b, p, k):
      pltpu.make_async_copy(feat_hbm.at[srci[p].at[k]], gdst[b], sgs[b]).wait()

    def scatter_start(b, p, k):
      pltpu.async_copy(rows[b], agg_sh.at[dsti[p].at[k]], sss[b], add=True)

    def scatter_drain(b):
      pltpu.make_async_copy(rows[b], agg_sh.at[dsti[0].at[0]], sss[b]).wait()

    def chunk_ops(c, g_of, drain, first_group):
      k, lg, b = c % SG, c // SG, (c if first_group else SG + c) % ring
      p = (lg if first_group else 1 + lg) % 2
      if drain:
        scatter_drain(b)
      if k == 0:
        slab_wait(p, g_of(lg))
      gather_start(b, p, k)
      c2 = c - 2
      if c2 >= 0 or not first_group:
        k2, lg2 = c2 % SG, c2 // SG
        b2 = (c2 if first_group else SG + c2) % ring
        p2 = (lg2 if first_group else 1 + lg2) % 2
        gather_wait(b2, p2, k2)
        scatter_start(b2, p2, k2)
      if k == ring - 1:
        slab_start(1 - p, g_of(lg + 1))

    slab_start(0, 0)
    plsc.subcore_barrier()

    for c in range(SG):
      chunk_ops(c, lambda lg: jnp.int32(lg), drain=(c >= ring),
                first_group=True)

    gpi = unit // SG

    def step(i, carry):
      def g_of(lg):
        return jnp.minimum(1 + gpi * i + lg, NSG - 1)
      for c in range(unit):
        chunk_ops(c, g_of, drain=True, first_group=False)
      return carry

    lax.fori_loop(0, nit, step, 0)

    for t in (NCHUNK - 2, NCHUNK - 1):
      k2, b2 = t % SG, t % ring
      gather_wait(b2, 0, k2)
      scatter_start(b2, 0, k2)
    for t in range(NCHUNK - ring, NCHUNK):
      scatter_drain(t % ring)
    slab_wait(1, NSG - 1)
    plsc.subcore_barrier()

    def wstep(i, carry):
      ro = pl.multiple_of(roff + i * CK, 8)
      pltpu.sync_copy(agg_sh.at[pl.ds(ro, CK)], rows[0].at[pl.ds(0, CK)])
      pltpu.sync_copy(rows[0].at[pl.ds(0, CK)], agg_out.at[cid, pl.ds(ro, CK)])
      return carry

    lax.fori_loop(0, nch, wstep, 0)

  return pl.kernel(
      body, out_type=out_type, mesh=mesh, scratch_types=scratch,
      compiler_params=pltpu.CompilerParams(use_tc_tiling_on_sc=False))


_sc_agg_cnt = _make_sc_agg(DP, 3)
_sc_agg = _make_sc_agg(D, 4)

BN = 1000


def _dense1_body(agg_ref, x_ref, wl_ref, b_ref, wr_ref, out_ref):
  a = agg_ref[0] + agg_ref[1]
  cnt = a[:, D:D + 1]
  mean = a[:, :D] / jnp.maximum(cnt, 1.0)
  h = (lax.dot_general(mean, wl_ref[...], (((1,), (1,)), ((), ())),
                       preferred_element_type=jnp.float32)
       + b_ref[...]
       + lax.dot_general(x_ref[...], wr_ref[...], (((1,), (1,)), ((), ())),
                         preferred_element_type=jnp.float32))
  nrm = jnp.sqrt(jnp.sum(h * h, axis=-1, keepdims=True))
  h = h / jnp.maximum(nrm, 1e-12)
  out_ref[...] = jnp.maximum(h, 0.0)


def _dense2_body(agg_ref, cnt_ref, x_ref, wl_ref, b_ref, wr_ref, out_ref):
  a = agg_ref[0] + agg_ref[1]
  cnt = cnt_ref[0, :, 0:1] + cnt_ref[1, :, 0:1]
  mean = a / jnp.maximum(cnt, 1.0)
  h = (lax.dot_general(mean, wl_ref[...], (((1,), (1,)), ((), ())),
                       preferred_element_type=jnp.float32)
       + b_ref[...]
       + lax.dot_general(x_ref[...], wr_ref[...], (((1,), (1,)), ((), ())),
                         preferred_element_type=jnp.float32))
  nrm = jnp.sqrt(jnp.sum(h * h, axis=-1, keepdims=True))
  out_ref[...] = h / jnp.maximum(nrm, 1e-12)


def kernel(x, edge_index, W1l, b1, W1r, W2l, b2, W2r):
  xp = jnp.concatenate([x, jnp.ones((N, CW), jnp.float32)], axis=1)
  (aggx,) = _sc_agg_cnt(xp, edge_index)
  wspec = pl.BlockSpec((D, D), lambda i: (0, 0))
  bspec = pl.BlockSpec((1, D), lambda i: (0, 0))
  h1 = pl.pallas_call(
      _dense1_body,
      grid=(N // BN,),
      in_specs=[
          pl.BlockSpec((NC, BN, DP), lambda i: (0, i, 0)),
          pl.BlockSpec((BN, D), lambda i: (i, 0)),
          wspec, bspec, wspec,
      ],
      out_specs=pl.BlockSpec((BN, D), lambda i: (i, 0)),
      out_shape=jax.ShapeDtypeStruct((N, D), jnp.float32),
  )(aggx, x, W1l, b1.reshape(1, D), W1r)
  (agg2,) = _sc_agg(h1, edge_index)
  return pl.pallas_call(
      _dense2_body,
      grid=(N // BN,),
      in_specs=[
          pl.BlockSpec((NC, BN, D), lambda i: (0, i, 0)),
          pl.BlockSpec((NC, BN, CW), lambda i: (0, i, 0)),
          pl.BlockSpec((BN, D), lambda i: (i, 0)),
          wspec, bspec, wspec,
      ],
      out_specs=pl.BlockSpec((BN, D), lambda i: (i, 0)),
      out_shape=jax.ShapeDtypeStruct((N, D), jnp.float32),
  )(agg2, aggx[:, :, D:], h1, W2l, b2.reshape(1, D), W2r)
